# baseline (speedup 1.0000x reference)
import functools

import jax
import jax.numpy as jnp
from jax.experimental import pallas as pl
from jax.experimental.pallas import tpu as pltpu


def _add_block(x_ref, pos_ref, o_ref):
    o_ref[...] = x_ref[...] + pos_ref[...][None, :, :]


@functools.partial(jax.jit, static_argnames=())
def kernel(x, pos_table):
    batch, seq_len, d = x.shape
    s_blk = 512
    grid = (seq_len // s_blk,)
    return pl.pallas_call(
        _add_block,
        grid=grid,
        in_specs=[
            pl.BlockSpec((batch, s_blk, d), lambda s: (0, s, 0)),
            pl.BlockSpec((s_blk, d), lambda s: (s, 0)),
        ],
        out_specs=pl.BlockSpec((batch, s_blk, d), lambda s: (0, s, 0)),
        out_shape=jax.ShapeDtypeStruct((batch, seq_len, d), x.dtype),
        compiler_params=pltpu.CompilerParams(
            dimension_semantics=("parallel",),
        ),
    )(x, pos_table[:seq_len])
